# Initial kernel scaffold; baseline (speedup 1.0000x reference)
import jax
import jax.numpy as jnp
from jax.experimental import pallas as pl


def kernel(x, edge_index, W, b, gamma, beta):
    raise NotImplementedError("write your pallas kernel here")


# speedup vs baseline: 7.1232x; 7.1232x over previous
import functools

import jax
import jax.numpy as jnp
from jax import lax
from jax.experimental import pallas as pl
from jax.experimental.pallas import tpu as pltpu
from jax.experimental.pallas import tpu_sc as plsc

N_NODES = 10000
N_EDGES = 160000
CH = 256
HALF = 128
EPS = 1e-5

E_PAD = 163840
CHUNK = 128
NPAD = 12800
ROWS_PER_TILE = 800

_MESH = plsc.VectorSubcoreMesh(core_axis_name="c", subcore_axis_name="s")


@functools.partial(
    pl.kernel,
    out_type=jax.ShapeDtypeStruct((2 * NPAD, 16), jnp.float32),
    mesh=_MESH,
    scratch_types=[
        pltpu.VMEM((1, CHUNK), jnp.int32),
        pltpu.VMEM((CHUNK, 16), jnp.float32),
        pltpu.VMEM_SHARED((NPAD, 16), jnp.float32),
    ],
)
def _sc_deg(dst_hbm, ones_hbm, zeros_hbm, out_hbm, dst_v, ones_v, deg_sh):
    cid = lax.axis_index("c")
    sid = lax.axis_index("s")
    pltpu.sync_copy(zeros_hbm, deg_sh.at[pl.ds(sid * ROWS_PER_TILE, ROWS_PER_TILE)])
    pltpu.sync_copy(ones_hbm, ones_v)
    plsc.subcore_barrier()

    n_chunks = E_PAD // 2 // 16 // CHUNK
    tile_base = cid * (E_PAD // 2) + sid * (n_chunks * CHUNK)

    def body(k, carry):
        e0 = tile_base + k * CHUNK
        pltpu.sync_copy(dst_hbm.at[pl.ds(e0, CHUNK)], dst_v.at[0])
        pltpu.sync_copy(ones_v, deg_sh.at[dst_v.at[0]], add=True)
        return carry

    lax.fori_loop(0, n_chunks, body, 0)
    plsc.subcore_barrier()
    row0 = sid * ROWS_PER_TILE
    pltpu.sync_copy(
        deg_sh.at[pl.ds(row0, ROWS_PER_TILE)],
        out_hbm.at[pl.ds(cid * NPAD + row0, ROWS_PER_TILE)],
    )


@functools.partial(
    pl.kernel,
    out_type=jax.ShapeDtypeStruct((2 * NPAD, HALF), jnp.float32),
    mesh=_MESH,
    scratch_types=[
        pltpu.VMEM((CHUNK,), jnp.int32),
        pltpu.VMEM((1, CHUNK), jnp.int32),
        pltpu.VMEM((CHUNK, HALF), jnp.float32),
        pltpu.VMEM_SHARED((NPAD, HALF), jnp.float32),
        pltpu.SemaphoreType.DMA,
    ],
)
def _sc_agg(xs_hbm, src_hbm, dst_hbm, zeros_hbm, out_hbm,
            src_v, dst_v, rows_v, agg_sh, sem):
    cid = lax.axis_index("c")
    sid = lax.axis_index("s")
    pltpu.sync_copy(zeros_hbm, agg_sh.at[pl.ds(sid * ROWS_PER_TILE, ROWS_PER_TILE)])
    plsc.subcore_barrier()

    n_chunks = E_PAD // 16 // CHUNK
    tile_base = cid * E_PAD + sid * (n_chunks * CHUNK)
    dst_base = sid * (n_chunks * CHUNK)

    def body(k, carry):
        pltpu.sync_copy(src_hbm.at[pl.ds(tile_base + k * CHUNK, CHUNK)], src_v)
        pltpu.sync_copy(dst_hbm.at[pl.ds(dst_base + k * CHUNK, CHUNK)], dst_v.at[0])
        pltpu.async_copy(xs_hbm.at[src_v], rows_v, sem).wait()
        pltpu.sync_copy(rows_v, agg_sh.at[dst_v.at[0]], add=True)
        return carry

    lax.fori_loop(0, n_chunks, body, 0)
    plsc.subcore_barrier()
    row0 = sid * ROWS_PER_TILE
    pltpu.sync_copy(
        agg_sh.at[pl.ds(row0, ROWS_PER_TILE)],
        out_hbm.at[pl.ds(cid * NPAD + row0, ROWS_PER_TILE)],
    )


ROWB = 400
N_RB = N_NODES // ROWB
NPAD_RB = NPAD // ROWB


def _mm_body(deg_lo, deg_hi, x_ref, w_ref, out_ref):
    d = deg_lo[:, 0:1] + deg_hi[:, 0:1] + 1.0
    dis = lax.rsqrt(d)
    xw = lax.dot_general(
        x_ref[...], w_ref[...], (((1,), (1,)), ((), ())),
        preferred_element_type=jnp.float32,
    )
    out_ref[...] = xw * dis


def _tc_mm(deg2, x, W):
    return pl.pallas_call(
        _mm_body,
        grid=(N_RB, 2),
        in_specs=[
            pl.BlockSpec((ROWB, 16), lambda i, j: (i, 0)),
            pl.BlockSpec((ROWB, 16), lambda i, j: (NPAD_RB + i, 0)),
            pl.BlockSpec((ROWB, CH), lambda i, j: (i, 0)),
            pl.BlockSpec((HALF, CH), lambda i, j: (j, 0)),
        ],
        out_specs=pl.BlockSpec((ROWB, HALF), lambda i, j: (j * N_RB + i, 0)),
        out_shape=jax.ShapeDtypeStruct((2 * N_NODES, HALF), jnp.float32),
    )(deg2, deg2, x, W)


def _out0(deg_lo, deg_hi, agg_lo, agg_hi, xs_lo, xs_hi, b_ref):
    d = deg_lo[:, 0:1] + deg_hi[:, 0:1] + 1.0
    dis = lax.rsqrt(d)
    agg = jnp.concatenate([agg_lo[...], agg_hi[...]], axis=1)
    xs = jnp.concatenate([xs_lo[...], xs_hi[...]], axis=1)
    return dis * (agg + xs) + b_ref[...]


def _stats_body(deg_lo, deg_hi, agg_lo, agg_hi, xs_lo, xs_hi, b_ref, stats_ref):
    out0 = _out0(deg_lo, deg_hi, agg_lo, agg_hi, xs_lo, xs_hi, b_ref)
    s = jnp.sum(out0, axis=0, keepdims=True)
    q = jnp.sum(out0 * out0, axis=0, keepdims=True)

    @pl.when(pl.program_id(0) == 0)
    def _():
        stats_ref[...] = jnp.zeros_like(stats_ref)

    stats_ref[...] += jnp.concatenate([s, q], axis=0)


def _bn_body(deg_lo, deg_hi, agg_lo, agg_hi, xs_lo, xs_hi, b_ref,
             stats_ref, g_ref, be_ref, out_ref):
    out0 = _out0(deg_lo, deg_hi, agg_lo, agg_hi, xs_lo, xs_hi, b_ref)
    inv_n = 1.0 / N_NODES
    mean = stats_ref[0:1, :] * inv_n
    var = stats_ref[1:2, :] * inv_n - mean * mean
    inv = lax.rsqrt(var + EPS)
    out_ref[...] = jnp.maximum((out0 - mean) * inv * g_ref[...] + be_ref[...], 0.0)


_COMMON_SPECS = [
    pl.BlockSpec((ROWB, 16), lambda i: (i, 0)),
    pl.BlockSpec((ROWB, 16), lambda i: (NPAD_RB + i, 0)),
    pl.BlockSpec((ROWB, HALF), lambda i: (i, 0)),
    pl.BlockSpec((ROWB, HALF), lambda i: (NPAD_RB + i, 0)),
    pl.BlockSpec((ROWB, HALF), lambda i: (i, 0)),
    pl.BlockSpec((ROWB, HALF), lambda i: (N_RB + i, 0)),
    pl.BlockSpec((1, CH), lambda i: (0, 0)),
]


def _tc_stats(deg2, agg, xs, b2):
    return pl.pallas_call(
        _stats_body,
        grid=(N_RB,),
        in_specs=list(_COMMON_SPECS),
        out_specs=pl.BlockSpec((2, CH), lambda i: (0, 0)),
        out_shape=jax.ShapeDtypeStruct((2, CH), jnp.float32),
    )(deg2, deg2, agg, agg, xs, xs, b2)


def _tc_bn(deg2, agg, xs, b2, stats, g2, be2):
    return pl.pallas_call(
        _bn_body,
        grid=(N_RB,),
        in_specs=list(_COMMON_SPECS) + [
            pl.BlockSpec((2, CH), lambda i: (0, 0)),
            pl.BlockSpec((1, CH), lambda i: (0, 0)),
            pl.BlockSpec((1, CH), lambda i: (0, 0)),
        ],
        out_specs=pl.BlockSpec((ROWB, CH), lambda i: (i, 0)),
        out_shape=jax.ShapeDtypeStruct((N_NODES, CH), jnp.float32),
    )(deg2, deg2, agg, agg, xs, xs, b2, stats, g2, be2)


def kernel(x, edge_index, W, b, gamma, beta):
    src = edge_index[0].astype(jnp.int32)
    dst = edge_index[1].astype(jnp.int32)
    pad = E_PAD - N_EDGES
    src_p = jnp.concatenate([src, jnp.zeros((pad,), jnp.int32)])
    dst_p = jnp.concatenate([dst, jnp.full((pad,), N_NODES, jnp.int32)])
    src2 = jnp.concatenate([src_p, src_p + N_NODES])

    ones_a = jnp.ones((CHUNK, 16), jnp.float32)
    zeros_a = jnp.zeros((ROWS_PER_TILE, 16), jnp.float32)
    zeros_b = jnp.zeros((ROWS_PER_TILE, HALF), jnp.float32)
    b2 = b.reshape(1, CH)
    g2 = gamma.reshape(1, CH)
    be2 = beta.reshape(1, CH)

    deg2 = _sc_deg(dst_p, ones_a, zeros_a)
    xs = _tc_mm(deg2, x, W)
    agg = _sc_agg(xs, src2, dst_p, zeros_b)
    stats = _tc_stats(deg2, agg, xs, b2)
    return _tc_bn(deg2, agg, xs, b2, stats, g2, be2)


# speedup vs baseline: 7.4470x; 1.0455x over previous
import functools

import jax
import jax.numpy as jnp
from jax import lax
from jax.experimental import pallas as pl
from jax.experimental.pallas import tpu as pltpu
from jax.experimental.pallas import tpu_sc as plsc

N_NODES = 10000
N_EDGES = 160000
CH = 256
HALF = 128
EPS = 1e-5

E_PAD = 163840
CHUNK = 128
NPAD = 10112
ROWS_PER_TILE = 632

_MESH = plsc.VectorSubcoreMesh(core_axis_name="c", subcore_axis_name="s")


@functools.partial(
    pl.kernel,
    out_type=jax.ShapeDtypeStruct((2 * NPAD, HALF), jnp.float32),
    mesh=_MESH,
    scratch_types=[
        pltpu.VMEM((1, CHUNK), jnp.int32),
        pltpu.VMEM((CHUNK, HALF), jnp.float32),
        pltpu.VMEM_SHARED((NPAD, HALF), jnp.float32),
    ],
)
def _sc_deg(dst_hbm, ones_hbm, zeros_hbm, out_hbm, dst_v, ones_v, deg_sh):
    cid = lax.axis_index("c")
    sid = lax.axis_index("s")
    pltpu.sync_copy(zeros_hbm, deg_sh.at[pl.ds(sid * ROWS_PER_TILE, ROWS_PER_TILE)])
    pltpu.sync_copy(ones_hbm, ones_v)
    plsc.subcore_barrier()

    n_chunks = E_PAD // 2 // 16 // CHUNK
    tile_base = (cid * 16 + sid) * (n_chunks * CHUNK)

    def body(k, carry):
        pltpu.sync_copy(dst_hbm.at[pl.ds(tile_base + k * CHUNK, CHUNK)],
                        dst_v.at[0])
        pltpu.sync_copy(ones_v, deg_sh.at[dst_v.at[0]], add=True)
        return carry

    lax.fori_loop(0, n_chunks, body, 0)
    plsc.subcore_barrier()
    row0 = sid * ROWS_PER_TILE
    pltpu.sync_copy(
        deg_sh.at[pl.ds(row0, ROWS_PER_TILE)],
        out_hbm.at[pl.ds(cid * NPAD + row0, ROWS_PER_TILE)],
    )


N_CHUNKS = E_PAD // 16 // CHUNK


@functools.partial(
    pl.kernel,
    out_type=jax.ShapeDtypeStruct((2 * NPAD, HALF), jnp.float32),
    mesh=_MESH,
    scratch_types=[
        pltpu.VMEM((CHUNK,), jnp.int32),
        pltpu.VMEM((1, CHUNK), jnp.int32),
        pltpu.VMEM((CHUNK, HALF), jnp.float32),
        pltpu.VMEM_SHARED((NPAD, HALF), jnp.float32),
        pltpu.SemaphoreType.DMA,
    ],
)
def _sc_agg(xs_hbm, src_hbm, dst_hbm, zeros_hbm, out_hbm,
            src_v, dst_v, buf0, agg_sh, sem0):
    cid = lax.axis_index("c")
    sid = lax.axis_index("s")
    pltpu.sync_copy(zeros_hbm, agg_sh.at[pl.ds(sid * ROWS_PER_TILE, ROWS_PER_TILE)])
    plsc.subcore_barrier()

    tile_base = cid * E_PAD + sid * (N_CHUNKS * CHUNK)
    dst_base = sid * (N_CHUNKS * CHUNK)

    def body(k, carry):
        pltpu.sync_copy(src_hbm.at[pl.ds(tile_base + k * CHUNK, CHUNK)], src_v)
        pltpu.sync_copy(dst_hbm.at[pl.ds(dst_base + k * CHUNK, CHUNK)],
                        dst_v.at[0])
        pltpu.async_copy(xs_hbm.at[src_v], buf0, sem0).wait()
        pltpu.sync_copy(buf0, agg_sh.at[dst_v.at[0]], add=True)
        return carry

    lax.fori_loop(0, N_CHUNKS, body, 0)

    plsc.subcore_barrier()
    out0 = sid * ROWS_PER_TILE
    pltpu.sync_copy(
        agg_sh.at[pl.ds(out0, ROWS_PER_TILE)],
        out_hbm.at[pl.ds(cid * NPAD + out0, ROWS_PER_TILE)],
    )


ROWB = 400
N_RB = N_NODES // ROWB


def _mm_body(deg_lo, deg_hi, x_ref, w_ref, out_ref):
    d = deg_lo[0, :, 0:1] + deg_hi[0, :, 0:1] + 1.0
    dis = lax.rsqrt(d)
    xw = lax.dot_general(
        x_ref[...], w_ref[...], (((1,), (1,)), ((), ())),
        preferred_element_type=jnp.float32,
    )
    out_ref[...] = xw * dis


def _tc_mm(deg2, x, W):
    return pl.pallas_call(
        _mm_body,
        grid=(N_RB, 2),
        in_specs=[
            pl.BlockSpec((1, ROWB, HALF), lambda i, j: (0, i, 0)),
            pl.BlockSpec((1, ROWB, HALF), lambda i, j: (1, i, 0)),
            pl.BlockSpec((ROWB, CH), lambda i, j: (i, 0)),
            pl.BlockSpec((HALF, CH), lambda i, j: (j, 0)),
        ],
        out_specs=pl.BlockSpec((ROWB, HALF), lambda i, j: (j * N_RB + i, 0)),
        out_shape=jax.ShapeDtypeStruct((2 * N_NODES, HALF), jnp.float32),
    )(deg2, deg2, x, W)


def _out0(deg_lo, deg_hi, agg_lo, agg_hi, xs_lo, xs_hi, b_ref):
    d = deg_lo[0, :, 0:1] + deg_hi[0, :, 0:1] + 1.0
    dis = lax.rsqrt(d)
    agg = jnp.concatenate([agg_lo[0], agg_hi[0]], axis=1)
    xs = jnp.concatenate([xs_lo[...], xs_hi[...]], axis=1)
    return dis * (agg + xs) + b_ref[...]


def _stats_body(deg_lo, deg_hi, agg_lo, agg_hi, xs_lo, xs_hi, b_ref, stats_ref):
    out0 = _out0(deg_lo, deg_hi, agg_lo, agg_hi, xs_lo, xs_hi, b_ref)
    s = jnp.sum(out0, axis=0, keepdims=True)
    q = jnp.sum(out0 * out0, axis=0, keepdims=True)

    @pl.when(pl.program_id(0) == 0)
    def _():
        stats_ref[...] = jnp.zeros_like(stats_ref)

    stats_ref[...] += jnp.concatenate([s, q], axis=0)


def _bn_body(deg_lo, deg_hi, agg_lo, agg_hi, xs_lo, xs_hi, b_ref,
             stats_ref, g_ref, be_ref, out_ref):
    out0 = _out0(deg_lo, deg_hi, agg_lo, agg_hi, xs_lo, xs_hi, b_ref)
    inv_n = 1.0 / N_NODES
    mean = stats_ref[0:1, :] * inv_n
    var = stats_ref[1:2, :] * inv_n - mean * mean
    inv = lax.rsqrt(var + EPS)
    out_ref[...] = jnp.maximum((out0 - mean) * inv * g_ref[...] + be_ref[...], 0.0)


_COMMON_SPECS = [
    pl.BlockSpec((1, ROWB, HALF), lambda i: (0, i, 0)),
    pl.BlockSpec((1, ROWB, HALF), lambda i: (1, i, 0)),
    pl.BlockSpec((1, ROWB, HALF), lambda i: (0, i, 0)),
    pl.BlockSpec((1, ROWB, HALF), lambda i: (1, i, 0)),
    pl.BlockSpec((ROWB, HALF), lambda i: (i, 0)),
    pl.BlockSpec((ROWB, HALF), lambda i: (N_RB + i, 0)),
    pl.BlockSpec((1, CH), lambda i: (0, 0)),
]


def _tc_stats(deg2, agg, xs, b2):
    return pl.pallas_call(
        _stats_body,
        grid=(N_RB,),
        in_specs=list(_COMMON_SPECS),
        out_specs=pl.BlockSpec((2, CH), lambda i: (0, 0)),
        out_shape=jax.ShapeDtypeStruct((2, CH), jnp.float32),
    )(deg2, deg2, agg, agg, xs, xs, b2)


def _tc_bn(deg2, agg, xs, b2, stats, g2, be2):
    return pl.pallas_call(
        _bn_body,
        grid=(N_RB,),
        in_specs=list(_COMMON_SPECS) + [
            pl.BlockSpec((2, CH), lambda i: (0, 0)),
            pl.BlockSpec((1, CH), lambda i: (0, 0)),
            pl.BlockSpec((1, CH), lambda i: (0, 0)),
        ],
        out_specs=pl.BlockSpec((ROWB, CH), lambda i: (i, 0)),
        out_shape=jax.ShapeDtypeStruct((N_NODES, CH), jnp.float32),
    )(deg2, deg2, agg, agg, xs, xs, b2, stats, g2, be2)


def kernel(x, edge_index, W, b, gamma, beta):
    src = edge_index[0].astype(jnp.int32)
    dst = edge_index[1].astype(jnp.int32)
    pad = E_PAD - N_EDGES
    src_p = jnp.concatenate([src, jnp.zeros((pad,), jnp.int32)])
    dst_p = jnp.concatenate([dst, jnp.full((pad,), N_NODES, jnp.int32)])
    src2 = jnp.concatenate([src_p, src_p + N_NODES])

    ones_a = jnp.ones((CHUNK, HALF), jnp.float32)
    zeros_a = jnp.zeros((ROWS_PER_TILE, HALF), jnp.float32)
    zeros_b = jnp.zeros((ROWS_PER_TILE, HALF), jnp.float32)
    b2 = b.reshape(1, CH)
    g2 = gamma.reshape(1, CH)
    be2 = beta.reshape(1, CH)

    deg2 = _sc_deg(dst_p, ones_a, zeros_a).reshape(2, NPAD, HALF)
    xs = _tc_mm(deg2, x, W)
    agg = _sc_agg(xs, src2, dst_p, zeros_b).reshape(2, NPAD, HALF)
    stats = _tc_stats(deg2, agg, xs, b2)
    return _tc_bn(deg2, agg, xs, b2, stats, g2, be2)


# speedup vs baseline: 8.3746x; 1.1246x over previous
import functools

import jax
import jax.numpy as jnp
from jax import lax
from jax.experimental import pallas as pl
from jax.experimental.pallas import tpu as pltpu
from jax.experimental.pallas import tpu_sc as plsc

N_NODES = 10000
N_EDGES = 160000
CH = 256
HALF = 128
EPS = 1e-5

E_PAD = 163840
CHUNK = 128
NPAD = 10112
ROWS_PER_TILE = 632

_MESH = plsc.VectorSubcoreMesh(core_axis_name="c", subcore_axis_name="s")


@functools.partial(
    pl.kernel,
    out_type=jax.ShapeDtypeStruct((2 * NPAD, HALF), jnp.float32),
    mesh=_MESH,
    scratch_types=[
        pltpu.VMEM((1, CHUNK), jnp.int32),
        pltpu.VMEM((CHUNK, HALF), jnp.float32),
        pltpu.VMEM_SHARED((NPAD, HALF), jnp.float32),
    ],
)
def _sc_deg(dst_hbm, ones_hbm, zeros_hbm, out_hbm, dst_v, ones_v, deg_sh):
    cid = lax.axis_index("c")
    sid = lax.axis_index("s")
    pltpu.sync_copy(zeros_hbm, deg_sh.at[pl.ds(sid * ROWS_PER_TILE, ROWS_PER_TILE)])
    pltpu.sync_copy(ones_hbm, ones_v)
    plsc.subcore_barrier()

    n_chunks = E_PAD // 2 // 16 // CHUNK
    tile_base = (cid * 16 + sid) * (n_chunks * CHUNK)

    def body(k, carry):
        pltpu.sync_copy(dst_hbm.at[pl.ds(tile_base + k * CHUNK, CHUNK)],
                        dst_v.at[0])
        pltpu.sync_copy(ones_v, deg_sh.at[dst_v.at[0]], add=True)
        return carry

    lax.fori_loop(0, n_chunks, body, 0)
    plsc.subcore_barrier()
    row0 = sid * ROWS_PER_TILE
    pltpu.sync_copy(
        deg_sh.at[pl.ds(row0, ROWS_PER_TILE)],
        out_hbm.at[pl.ds(cid * NPAD + row0, ROWS_PER_TILE)],
    )


N_CHUNKS = E_PAD // 16 // CHUNK


@functools.partial(
    pl.kernel,
    out_type=jax.ShapeDtypeStruct((2 * NPAD, HALF), jnp.float32),
    mesh=_MESH,
    scratch_types=[
        pltpu.VMEM((CHUNK,), jnp.int32),
        pltpu.VMEM((CHUNK,), jnp.int32),
        pltpu.VMEM((1, CHUNK), jnp.int32),
        pltpu.VMEM((1, CHUNK), jnp.int32),
        pltpu.VMEM((CHUNK, HALF), jnp.float32),
        pltpu.VMEM((CHUNK, HALF), jnp.float32),
        pltpu.VMEM_SHARED((NPAD, HALF), jnp.float32),
        pltpu.SemaphoreType.DMA,
        pltpu.SemaphoreType.DMA,
    ],
)
def _sc_agg(xs_hbm, src_hbm, dst_hbm, zeros_hbm, out_hbm,
            src_v0, src_v1, dst_v0, dst_v1, buf0, buf1, agg_sh, sem0, sem1):
    cid = lax.axis_index("c")
    sid = lax.axis_index("s")
    pltpu.sync_copy(zeros_hbm, agg_sh.at[pl.ds(sid * ROWS_PER_TILE, ROWS_PER_TILE)])
    plsc.subcore_barrier()

    tile_base = cid * E_PAD + sid * (N_CHUNKS * CHUNK)
    dst_base = sid * (N_CHUNKS * CHUNK)

    def idx_load(k, srcb, dstb):
        pltpu.sync_copy(src_hbm.at[pl.ds(tile_base + k * CHUNK, CHUNK)], srcb)
        pltpu.sync_copy(dst_hbm.at[pl.ds(dst_base + k * CHUNK, CHUNK)],
                        dstb.at[0])

    idx_load(0, src_v0, dst_v0)

    def body(j, carry):
        k0 = 2 * j
        k1 = 2 * j + 1
        d0 = pltpu.async_copy(xs_hbm.at[src_v0], buf0, sem0)
        idx_load(k1, src_v1, dst_v1)
        d0.wait()
        pltpu.sync_copy(buf0, agg_sh.at[dst_v0.at[0]], add=True)

        d1 = pltpu.async_copy(xs_hbm.at[src_v1], buf1, sem1)

        @pl.when(k0 + 2 < N_CHUNKS)
        def _():
            idx_load(k0 + 2, src_v0, dst_v0)

        d1.wait()
        pltpu.sync_copy(buf1, agg_sh.at[dst_v1.at[0]], add=True)
        return carry

    lax.fori_loop(0, N_CHUNKS // 2, body, 0)

    plsc.subcore_barrier()
    out0 = sid * ROWS_PER_TILE
    pltpu.sync_copy(
        agg_sh.at[pl.ds(out0, ROWS_PER_TILE)],
        out_hbm.at[pl.ds(cid * NPAD + out0, ROWS_PER_TILE)],
    )


ROWB = 400
N_RB = N_NODES // ROWB


def _mm_body(deg_lo, deg_hi, x_ref, w_ref, out_ref):
    d = deg_lo[0, :, 0:1] + deg_hi[0, :, 0:1] + 1.0
    dis = lax.rsqrt(d)
    xw = lax.dot_general(
        x_ref[...], w_ref[...], (((1,), (1,)), ((), ())),
        preferred_element_type=jnp.float32,
    )
    out_ref[...] = xw * dis


def _tc_mm(deg2, x, W):
    return pl.pallas_call(
        _mm_body,
        grid=(N_RB, 2),
        in_specs=[
            pl.BlockSpec((1, ROWB, HALF), lambda i, j: (0, i, 0)),
            pl.BlockSpec((1, ROWB, HALF), lambda i, j: (1, i, 0)),
            pl.BlockSpec((ROWB, CH), lambda i, j: (i, 0)),
            pl.BlockSpec((HALF, CH), lambda i, j: (j, 0)),
        ],
        out_specs=pl.BlockSpec((ROWB, HALF), lambda i, j: (j * N_RB + i, 0)),
        out_shape=jax.ShapeDtypeStruct((2 * N_NODES, HALF), jnp.float32),
    )(deg2, deg2, x, W)


def _out0(deg_lo, deg_hi, agg_lo, agg_hi, xs_lo, xs_hi, b_ref):
    d = deg_lo[0, :, 0:1] + deg_hi[0, :, 0:1] + 1.0
    dis = lax.rsqrt(d)
    agg = jnp.concatenate([agg_lo[0], agg_hi[0]], axis=1)
    xs = jnp.concatenate([xs_lo[...], xs_hi[...]], axis=1)
    return dis * (agg + xs) + b_ref[...]


def _stats_body(deg_lo, deg_hi, agg_lo, agg_hi, xs_lo, xs_hi, b_ref, stats_ref):
    out0 = _out0(deg_lo, deg_hi, agg_lo, agg_hi, xs_lo, xs_hi, b_ref)
    s = jnp.sum(out0, axis=0, keepdims=True)
    q = jnp.sum(out0 * out0, axis=0, keepdims=True)

    @pl.when(pl.program_id(0) == 0)
    def _():
        stats_ref[...] = jnp.zeros_like(stats_ref)

    stats_ref[...] += jnp.concatenate([s, q], axis=0)


def _bn_body(deg_lo, deg_hi, agg_lo, agg_hi, xs_lo, xs_hi, b_ref,
             stats_ref, g_ref, be_ref, out_ref):
    out0 = _out0(deg_lo, deg_hi, agg_lo, agg_hi, xs_lo, xs_hi, b_ref)
    inv_n = 1.0 / N_NODES
    mean = stats_ref[0:1, :] * inv_n
    var = stats_ref[1:2, :] * inv_n - mean * mean
    inv = lax.rsqrt(var + EPS)
    out_ref[...] = jnp.maximum((out0 - mean) * inv * g_ref[...] + be_ref[...], 0.0)


_COMMON_SPECS = [
    pl.BlockSpec((1, ROWB, HALF), lambda i: (0, i, 0)),
    pl.BlockSpec((1, ROWB, HALF), lambda i: (1, i, 0)),
    pl.BlockSpec((1, ROWB, HALF), lambda i: (0, i, 0)),
    pl.BlockSpec((1, ROWB, HALF), lambda i: (1, i, 0)),
    pl.BlockSpec((ROWB, HALF), lambda i: (i, 0)),
    pl.BlockSpec((ROWB, HALF), lambda i: (N_RB + i, 0)),
    pl.BlockSpec((1, CH), lambda i: (0, 0)),
]


def _tc_stats(deg2, agg, xs, b2):
    return pl.pallas_call(
        _stats_body,
        grid=(N_RB,),
        in_specs=list(_COMMON_SPECS),
        out_specs=pl.BlockSpec((2, CH), lambda i: (0, 0)),
        out_shape=jax.ShapeDtypeStruct((2, CH), jnp.float32),
    )(deg2, deg2, agg, agg, xs, xs, b2)


def _tc_bn(deg2, agg, xs, b2, stats, g2, be2):
    return pl.pallas_call(
        _bn_body,
        grid=(N_RB,),
        in_specs=list(_COMMON_SPECS) + [
            pl.BlockSpec((2, CH), lambda i: (0, 0)),
            pl.BlockSpec((1, CH), lambda i: (0, 0)),
            pl.BlockSpec((1, CH), lambda i: (0, 0)),
        ],
        out_specs=pl.BlockSpec((ROWB, CH), lambda i: (i, 0)),
        out_shape=jax.ShapeDtypeStruct((N_NODES, CH), jnp.float32),
    )(deg2, deg2, agg, agg, xs, xs, b2, stats, g2, be2)


def kernel(x, edge_index, W, b, gamma, beta):
    src = edge_index[0].astype(jnp.int32)
    dst = edge_index[1].astype(jnp.int32)
    pad = E_PAD - N_EDGES
    src_p = jnp.concatenate([src, jnp.zeros((pad,), jnp.int32)])
    dst_p = jnp.concatenate([dst, jnp.full((pad,), N_NODES, jnp.int32)])
    src2 = jnp.concatenate([src_p, src_p + N_NODES])

    ones_a = jnp.ones((CHUNK, HALF), jnp.float32)
    zeros_a = jnp.zeros((ROWS_PER_TILE, HALF), jnp.float32)
    zeros_b = jnp.zeros((ROWS_PER_TILE, HALF), jnp.float32)
    b2 = b.reshape(1, CH)
    g2 = gamma.reshape(1, CH)
    be2 = beta.reshape(1, CH)

    deg2 = _sc_deg(dst_p, ones_a, zeros_a).reshape(2, NPAD, HALF)
    xs = _tc_mm(deg2, x, W)
    agg = _sc_agg(xs, src2, dst_p, zeros_b).reshape(2, NPAD, HALF)
    stats = _tc_stats(deg2, agg, xs, b2)
    return _tc_bn(deg2, agg, xs, b2, stats, g2, be2)
